# baseline (speedup 1.0000x reference)
import functools

import jax
import jax.numpy as jnp
from jax import lax
from jax.experimental import pallas as pl
from jax.experimental.pallas import tpu as pltpu, tpu_sc as plsc

N_NODES = 10000
N_EDGES = 320000
D = 128

NC = 2
NS = 16
NW = NC * NS

H_CHUNK = 125
H_CHUNKS = (2 * N_EDGES) // NW // H_CHUNK
N_PAD = 640 * NS

_sc_mesh = plsc.VectorSubcoreMesh(core_axis_name="c", subcore_axis_name="s")


@functools.partial(
    pl.kernel,
    out_type=jax.ShapeDtypeStruct((NC * N_PAD,), jnp.float32),
    mesh=_sc_mesh,
    scratch_types=[
        pltpu.VMEM((H_CHUNKS // 2, H_CHUNK), jnp.int32),
        pltpu.VMEM((640,), jnp.float32),
        pltpu.VMEM_SHARED((N_PAD,), jnp.float32),
    ],
)
def _hist_kernel(idx_hbm, out_hbm, idx_v, buf_v, hist_sh):
    cid = lax.axis_index("c")
    sid = lax.axis_index("s")
    wid = cid * NS + sid

    @pl.loop(0, 40)
    def _(i):
        buf_v[pl.ds(i * 16, 16)] = jnp.zeros((16,), jnp.float32)

    pltpu.sync_copy(buf_v, hist_sh.at[pl.ds(sid * 640, 640)])
    plsc.subcore_barrier()

    @pl.loop(0, 40)
    def _(i):
        buf_v[pl.ds(i * 16, 16)] = jnp.ones((16,), jnp.float32)

    for h in range(2):
        pltpu.sync_copy(idx_hbm.at[wid, h], idx_v)

        @pl.loop(0, H_CHUNKS // 2)
        def _(j):
            pltpu.sync_copy(buf_v.at[pl.ds(0, H_CHUNK)],
                            hist_sh.at[idx_v.at[j]], add=True)

    plsc.subcore_barrier()
    pltpu.sync_copy(hist_sh.at[pl.ds(sid * 640, 640)], buf_v)
    pltpu.sync_copy(buf_v, out_hbm.at[pl.ds(cid * N_PAD + sid * 640, 640)])


E_CHUNK = 125
E_CHUNKS = N_EDGES // NW // E_CHUNK
ROWS_PER_TILE = N_PAD // NS
CP_CHUNK = 80


@functools.partial(
    pl.kernel,
    out_type=jax.ShapeDtypeStruct((NC, N_PAD, D), jnp.float32),
    mesh=_sc_mesh,
    scratch_types=[
        pltpu.VMEM((E_CHUNKS // 2, E_CHUNK), jnp.int32),
        pltpu.VMEM((E_CHUNKS // 2, E_CHUNK), jnp.int32),
        pltpu.VMEM((E_CHUNK, D), jnp.float32),
        pltpu.VMEM((E_CHUNK, D), jnp.float32),
        pltpu.VMEM_SHARED((N_PAD, D), jnp.float32),
        pltpu.SemaphoreType.DMA,
        pltpu.SemaphoreType.DMA,
        pltpu.SemaphoreType.DMA,
        pltpu.SemaphoreType.DMA,
    ],
)
def _agg_kernel(y_hbm, src_hbm, dst_hbm, out_hbm,
                src_v, dst_v, rows0_v, rows1_v, agg_sh,
                sem0, sem1, ssem0, ssem1):
    cid = lax.axis_index("c")
    sid = lax.axis_index("s")
    wid = cid * NS + sid

    @pl.loop(0, E_CHUNK)
    def _(r):
        @pl.loop(0, D // 16)
        def _(c):
            rows0_v[r, pl.ds(c * 16, 16)] = jnp.zeros((16,), jnp.float32)

    @pl.loop(0, ROWS_PER_TILE // CP_CHUNK)
    def _(k):
        pltpu.sync_copy(
            rows0_v.at[pl.ds(0, CP_CHUNK)],
            agg_sh.at[pl.ds(sid * ROWS_PER_TILE + k * CP_CHUNK, CP_CHUNK)])

    plsc.subcore_barrier()

    HALF = E_CHUNKS // 2

    def _gather(j, buf, sem):
        pltpu.async_copy(y_hbm.at[src_v.at[j]], buf, sem)

    def _gather_wait(j, buf, sem):
        pltpu.make_async_copy(y_hbm.at[src_v.at[j]], buf, sem).wait()

    def _scatter(j, buf, sem):
        pltpu.async_copy(buf, agg_sh.at[dst_v.at[j]], sem, add=True)

    def _scatter_wait(j, buf, sem):
        pltpu.make_async_copy(buf, agg_sh.at[dst_v.at[j]], sem).wait()

    for h in range(2):
        pltpu.sync_copy(src_hbm.at[wid, h], src_v)
        pltpu.sync_copy(dst_hbm.at[wid, h], dst_v)

        _gather(0, rows0_v, sem0)
        _gather(1, rows1_v, sem1)

        @pl.loop(0, HALF // 2 - 1)
        def _(p):
            _gather_wait(2 * p, rows0_v, sem0)
            _scatter(2 * p, rows0_v, ssem0)
            _gather_wait(2 * p + 1, rows1_v, sem1)
            _scatter(2 * p + 1, rows1_v, ssem1)
            _scatter_wait(2 * p, rows0_v, ssem0)
            _gather(2 * p + 2, rows0_v, sem0)
            _scatter_wait(2 * p + 1, rows1_v, ssem1)
            _gather(2 * p + 3, rows1_v, sem1)

        _gather_wait(HALF - 2, rows0_v, sem0)
        _scatter(HALF - 2, rows0_v, ssem0)
        _gather_wait(HALF - 1, rows1_v, sem1)
        _scatter(HALF - 1, rows1_v, ssem1)
        _scatter_wait(HALF - 2, rows0_v, ssem0)
        _scatter_wait(HALF - 1, rows1_v, ssem1)

    plsc.subcore_barrier()

    base = sid * ROWS_PER_TILE
    pltpu.sync_copy(agg_sh.at[pl.ds(base, ROWS_PER_TILE)],
                    out_hbm.at[cid, pl.ds(base, ROWS_PER_TILE)])


ROW_BLK = 1000


def _scale_body(h0_ref, h1_ref, x_ref, y_ref):
    deg = (h0_ref[...] + h1_ref[...]) * 0.5 + 1.0
    norm = lax.rsqrt(jnp.maximum(deg, 1e-12))
    y_ref[...] = x_ref[...] * norm


def _scale(h0, h1, x):
    return pl.pallas_call(
        _scale_body,
        out_shape=jax.ShapeDtypeStruct((N_NODES, D), jnp.float32),
        grid=(N_NODES // ROW_BLK,),
        in_specs=[
            pl.BlockSpec((ROW_BLK, 1), lambda i: (i, 0)),
            pl.BlockSpec((ROW_BLK, 1), lambda i: (i, 0)),
            pl.BlockSpec((ROW_BLK, D), lambda i: (i, 0)),
        ],
        out_specs=pl.BlockSpec((ROW_BLK, D), lambda i: (i, 0)),
    )(h0, h1, x)


def _final_body(h0_ref, h1_ref, x_ref, p0_ref, p1_ref, w_ref, b_ref, o_ref):
    deg = (h0_ref[...] + h1_ref[...]) * 0.5 + 1.0
    norm = lax.rsqrt(jnp.maximum(deg, 1e-12))
    agg = (p0_ref[...] + p1_ref[...]) * norm + x_ref[...] * (norm * norm)
    out = jnp.dot(agg, w_ref[...], preferred_element_type=jnp.float32)
    o_ref[...] = jnp.maximum(out + b_ref[...], 0.0)


def _final(h0, h1, x, p0, p1, W, b2):
    return pl.pallas_call(
        _final_body,
        out_shape=jax.ShapeDtypeStruct((N_NODES, D), jnp.float32),
        grid=(N_NODES // ROW_BLK,),
        in_specs=[
            pl.BlockSpec((ROW_BLK, 1), lambda i: (i, 0)),
            pl.BlockSpec((ROW_BLK, 1), lambda i: (i, 0)),
            pl.BlockSpec((ROW_BLK, D), lambda i: (i, 0)),
            pl.BlockSpec((ROW_BLK, D), lambda i: (i, 0)),
            pl.BlockSpec((ROW_BLK, D), lambda i: (i, 0)),
            pl.BlockSpec((D, D), lambda i: (0, 0)),
            pl.BlockSpec((1, D), lambda i: (0, 0)),
        ],
        out_specs=pl.BlockSpec((ROW_BLK, D), lambda i: (i, 0)),
    )(h0, h1, x, p0, p1, W, b2)


def kernel(x, edge_index, W, b):
    ei = edge_index.astype(jnp.int32)
    all_idx = ei.reshape(NW, 2, H_CHUNKS // 2, H_CHUNK)
    src_r = ei[0].reshape(NW, 2, E_CHUNKS // 2, E_CHUNK)
    dst_r = ei[1].reshape(NW, 2, E_CHUNKS // 2, E_CHUNK)

    hist = _hist_kernel(all_idx)
    h0 = hist[:N_NODES].reshape(N_NODES, 1)
    h1 = hist[N_PAD:N_PAD + N_NODES].reshape(N_NODES, 1)

    y = _scale(h0, h1, x)
    parts = _agg_kernel(y, src_r, dst_r)

    return _final(h0, h1, x, parts[0, :N_NODES], parts[1, :N_NODES],
                  W, b.reshape(1, D))


# speedup vs baseline: 1.1815x; 1.1815x over previous
import functools

import jax
import jax.numpy as jnp
from jax import lax
from jax.experimental import pallas as pl
from jax.experimental.pallas import tpu as pltpu, tpu_sc as plsc

N_NODES = 10000
N_EDGES = 320000
D = 128

NC = 2
NS = 16
NW = NC * NS

H_CHUNK = 125
H_CHUNKS = (2 * N_EDGES) // NW // H_CHUNK
N_PAD = 640 * NS

_sc_mesh = plsc.VectorSubcoreMesh(core_axis_name="c", subcore_axis_name="s")


@functools.partial(
    pl.kernel,
    out_type=jax.ShapeDtypeStruct((NC * N_PAD,), jnp.float32),
    mesh=_sc_mesh,
    scratch_types=[
        pltpu.VMEM((H_CHUNKS // 2, H_CHUNK), jnp.int32),
        pltpu.VMEM((640,), jnp.float32),
        pltpu.VMEM_SHARED((N_PAD,), jnp.float32),
        pltpu.SemaphoreType.DMA,
    ],
)
def _hist_kernel(idx_hbm, out_hbm, idx_v, buf_v, hist_sh, hsem):
    cid = lax.axis_index("c")
    sid = lax.axis_index("s")
    wid = cid * NS + sid

    @pl.loop(0, 40)
    def _(i):
        buf_v[pl.ds(i * 16, 16)] = jnp.zeros((16,), jnp.float32)

    pltpu.sync_copy(buf_v, hist_sh.at[pl.ds(sid * 640, 640)])
    plsc.subcore_barrier()

    @pl.loop(0, 40)
    def _(i):
        buf_v[pl.ds(i * 16, 16)] = jnp.ones((16,), jnp.float32)

    GRP = 8
    for h in range(2):
        pltpu.sync_copy(idx_hbm.at[wid, h], idx_v)

        @pl.loop(0, H_CHUNKS // 2 // GRP)
        def _(g):
            for j in range(GRP):
                pltpu.async_copy(buf_v.at[pl.ds(0, H_CHUNK)],
                                 hist_sh.at[idx_v.at[g * GRP + j]], hsem,
                                 add=True)
            for j in range(GRP):
                pltpu.make_async_copy(
                    buf_v.at[pl.ds(0, H_CHUNK)],
                    hist_sh.at[idx_v.at[g * GRP + j]], hsem).wait()

    plsc.subcore_barrier()
    pltpu.sync_copy(hist_sh.at[pl.ds(sid * 640, 640)], buf_v)
    pltpu.sync_copy(buf_v, out_hbm.at[pl.ds(cid * N_PAD + sid * 640, 640)])


E_CHUNK = 125
E_CHUNKS = N_EDGES // NW // E_CHUNK
ROWS_PER_TILE = N_PAD // NS
CP_CHUNK = 80


@functools.partial(
    pl.kernel,
    out_type=jax.ShapeDtypeStruct((NC, N_PAD, D), jnp.float32),
    mesh=_sc_mesh,
    scratch_types=[
        pltpu.VMEM((E_CHUNKS // 2, E_CHUNK), jnp.int32),
        pltpu.VMEM((E_CHUNKS // 2, E_CHUNK), jnp.int32),
        pltpu.VMEM((E_CHUNK, D), jnp.float32),
        pltpu.VMEM((E_CHUNK, D), jnp.float32),
        pltpu.VMEM_SHARED((N_PAD, D), jnp.float32),
        pltpu.SemaphoreType.DMA,
        pltpu.SemaphoreType.DMA,
    ],
)
def _agg_kernel(y_hbm, src_hbm, dst_hbm, out_hbm,
                src_v, dst_v, rows0_v, rows1_v, agg_sh, sem0, sem1):
    cid = lax.axis_index("c")
    sid = lax.axis_index("s")
    wid = cid * NS + sid

    @pl.loop(0, E_CHUNK)
    def _(r):
        @pl.loop(0, D // 16)
        def _(c):
            rows0_v[r, pl.ds(c * 16, 16)] = jnp.zeros((16,), jnp.float32)

    @pl.loop(0, ROWS_PER_TILE // CP_CHUNK)
    def _(k):
        pltpu.sync_copy(
            rows0_v.at[pl.ds(0, CP_CHUNK)],
            agg_sh.at[pl.ds(sid * ROWS_PER_TILE + k * CP_CHUNK, CP_CHUNK)])

    plsc.subcore_barrier()

    HALF = E_CHUNKS // 2

    def _start(j, buf, sem):
        pltpu.async_copy(y_hbm.at[src_v.at[j]], buf, sem)

    def _finish(j, buf, sem):
        pltpu.make_async_copy(y_hbm.at[src_v.at[j]], buf, sem).wait()
        pltpu.sync_copy(buf, agg_sh.at[dst_v.at[j]], add=True)

    for h in range(2):
        pltpu.sync_copy(src_hbm.at[wid, h], src_v)
        pltpu.sync_copy(dst_hbm.at[wid, h], dst_v)

        _start(0, rows0_v, sem0)
        _start(1, rows1_v, sem1)

        @pl.loop(0, HALF // 2 - 1)
        def _(p):
            _finish(2 * p, rows0_v, sem0)
            _start(2 * p + 2, rows0_v, sem0)
            _finish(2 * p + 1, rows1_v, sem1)
            _start(2 * p + 3, rows1_v, sem1)

        _finish(HALF - 2, rows0_v, sem0)
        _finish(HALF - 1, rows1_v, sem1)

    plsc.subcore_barrier()

    base = sid * ROWS_PER_TILE
    pltpu.sync_copy(agg_sh.at[pl.ds(base, ROWS_PER_TILE)],
                    out_hbm.at[cid, pl.ds(base, ROWS_PER_TILE)])


ROW_BLK = 1000


def _scale_body(h0_ref, h1_ref, x_ref, y_ref):
    deg = (h0_ref[...] + h1_ref[...]) * 0.5 + 1.0
    norm = lax.rsqrt(jnp.maximum(deg, 1e-12))
    y_ref[...] = x_ref[...] * norm


def _scale(h0, h1, x):
    return pl.pallas_call(
        _scale_body,
        out_shape=jax.ShapeDtypeStruct((N_NODES, D), jnp.float32),
        grid=(N_NODES // ROW_BLK,),
        in_specs=[
            pl.BlockSpec((ROW_BLK, 1), lambda i: (i, 0)),
            pl.BlockSpec((ROW_BLK, 1), lambda i: (i, 0)),
            pl.BlockSpec((ROW_BLK, D), lambda i: (i, 0)),
        ],
        out_specs=pl.BlockSpec((ROW_BLK, D), lambda i: (i, 0)),
    )(h0, h1, x)


def _final_body(h0_ref, h1_ref, x_ref, p0_ref, p1_ref, w_ref, b_ref, o_ref):
    deg = (h0_ref[...] + h1_ref[...]) * 0.5 + 1.0
    norm = lax.rsqrt(jnp.maximum(deg, 1e-12))
    agg = (p0_ref[...] + p1_ref[...]) * norm + x_ref[...] * (norm * norm)
    out = jnp.dot(agg, w_ref[...], preferred_element_type=jnp.float32)
    o_ref[...] = jnp.maximum(out + b_ref[...], 0.0)


def _final(h0, h1, x, p0, p1, W, b2):
    return pl.pallas_call(
        _final_body,
        out_shape=jax.ShapeDtypeStruct((N_NODES, D), jnp.float32),
        grid=(N_NODES // ROW_BLK,),
        in_specs=[
            pl.BlockSpec((ROW_BLK, 1), lambda i: (i, 0)),
            pl.BlockSpec((ROW_BLK, 1), lambda i: (i, 0)),
            pl.BlockSpec((ROW_BLK, D), lambda i: (i, 0)),
            pl.BlockSpec((ROW_BLK, D), lambda i: (i, 0)),
            pl.BlockSpec((ROW_BLK, D), lambda i: (i, 0)),
            pl.BlockSpec((D, D), lambda i: (0, 0)),
            pl.BlockSpec((1, D), lambda i: (0, 0)),
        ],
        out_specs=pl.BlockSpec((ROW_BLK, D), lambda i: (i, 0)),
    )(h0, h1, x, p0, p1, W, b2)


def kernel(x, edge_index, W, b):
    ei = edge_index.astype(jnp.int32)
    all_idx = ei.reshape(NW, 2, H_CHUNKS // 2, H_CHUNK)
    src_r = ei[0].reshape(NW, 2, E_CHUNKS // 2, E_CHUNK)
    dst_r = ei[1].reshape(NW, 2, E_CHUNKS // 2, E_CHUNK)

    hist = _hist_kernel(all_idx)
    h0 = hist[:N_NODES].reshape(N_NODES, 1)
    h1 = hist[N_PAD:N_PAD + N_NODES].reshape(N_NODES, 1)

    y = _scale(h0, h1, x)
    parts = _agg_kernel(y, src_r, dst_r)

    return _final(h0, h1, x, parts[0, :N_NODES], parts[1, :N_NODES],
                  W, b.reshape(1, D))


# speedup vs baseline: 1.2583x; 1.0650x over previous
import functools

import jax
import jax.numpy as jnp
from jax import lax
from jax.experimental import pallas as pl
from jax.experimental.pallas import tpu as pltpu, tpu_sc as plsc

N_NODES = 10000
N_EDGES = 320000
D = 128

NC = 2
NS = 16
NW = NC * NS

H_CHUNK = 125
H_CHUNKS = (2 * N_EDGES) // NW // H_CHUNK
N_PAD = 640 * NS

_sc_mesh = plsc.VectorSubcoreMesh(core_axis_name="c", subcore_axis_name="s")


@functools.partial(
    pl.kernel,
    out_type=jax.ShapeDtypeStruct((NC * N_PAD,), jnp.float32),
    mesh=_sc_mesh,
    scratch_types=[
        pltpu.VMEM((H_CHUNKS // 4, H_CHUNK), jnp.int32),
        pltpu.VMEM((640,), jnp.float32),
        pltpu.VMEM_SHARED((N_PAD,), jnp.float32),
        pltpu.SemaphoreType.DMA,
    ],
)
def _hist_kernel(idx_hbm, out_hbm, idx_v, buf_v, hist_sh, hsem):
    cid = lax.axis_index("c")
    sid = lax.axis_index("s")
    wid = cid * NS + sid

    @pl.loop(0, 40)
    def _(i):
        buf_v[pl.ds(i * 16, 16)] = jnp.zeros((16,), jnp.float32)

    pltpu.sync_copy(buf_v, hist_sh.at[pl.ds(sid * 640, 640)])
    plsc.subcore_barrier()

    @pl.loop(0, 40)
    def _(i):
        buf_v[pl.ds(i * 16, 16)] = jnp.ones((16,), jnp.float32)

    GRP = 8
    for e in range(2):
        for h in range(2):
            pltpu.sync_copy(idx_hbm.at[e, wid, h], idx_v)

            @pl.loop(0, H_CHUNKS // 4 // GRP)
            def _(g):
                for j in range(GRP):
                    pltpu.async_copy(buf_v.at[pl.ds(0, H_CHUNK)],
                                     hist_sh.at[idx_v.at[g * GRP + j]],
                                     hsem, add=True)
                for j in range(GRP):
                    pltpu.make_async_copy(
                        buf_v.at[pl.ds(0, H_CHUNK)],
                        hist_sh.at[idx_v.at[g * GRP + j]], hsem).wait()

    plsc.subcore_barrier()
    pltpu.sync_copy(hist_sh.at[pl.ds(sid * 640, 640)], buf_v)
    pltpu.sync_copy(buf_v, out_hbm.at[pl.ds(cid * N_PAD + sid * 640, 640)])


E_CHUNK = 125
E_CHUNKS = N_EDGES // NW // E_CHUNK
ROWS_PER_TILE = N_PAD // NS
CP_CHUNK = 80


@functools.partial(
    pl.kernel,
    out_type=jax.ShapeDtypeStruct((NC, N_PAD, D), jnp.float32),
    mesh=_sc_mesh,
    scratch_types=[
        pltpu.VMEM((E_CHUNKS // 2, E_CHUNK), jnp.int32),
        pltpu.VMEM((E_CHUNKS // 2, E_CHUNK), jnp.int32),
        pltpu.VMEM((E_CHUNK, D), jnp.float32),
        pltpu.VMEM((E_CHUNK, D), jnp.float32),
        pltpu.VMEM_SHARED((N_PAD, D), jnp.float32),
        pltpu.SemaphoreType.DMA,
        pltpu.SemaphoreType.DMA,
    ],
)
def _agg_kernel(y_hbm, idx_hbm, out_hbm,
                src_v, dst_v, rows0_v, rows1_v, agg_sh, sem0, sem1):
    cid = lax.axis_index("c")
    sid = lax.axis_index("s")
    wid = cid * NS + sid

    @pl.loop(0, E_CHUNK)
    def _(r):
        @pl.loop(0, D // 16)
        def _(c):
            rows0_v[r, pl.ds(c * 16, 16)] = jnp.zeros((16,), jnp.float32)

    @pl.loop(0, ROWS_PER_TILE // CP_CHUNK)
    def _(k):
        pltpu.sync_copy(
            rows0_v.at[pl.ds(0, CP_CHUNK)],
            agg_sh.at[pl.ds(sid * ROWS_PER_TILE + k * CP_CHUNK, CP_CHUNK)])

    plsc.subcore_barrier()

    HALF = E_CHUNKS // 2

    def _start(j, buf, sem):
        pltpu.async_copy(y_hbm.at[src_v.at[j]], buf, sem)

    def _finish(j, buf, sem):
        pltpu.make_async_copy(y_hbm.at[src_v.at[j]], buf, sem).wait()
        pltpu.sync_copy(buf, agg_sh.at[dst_v.at[j]], add=True)

    for h in range(2):
        pltpu.sync_copy(idx_hbm.at[0, wid, h], src_v)
        pltpu.sync_copy(idx_hbm.at[1, wid, h], dst_v)

        _start(0, rows0_v, sem0)
        _start(1, rows1_v, sem1)

        @pl.loop(0, HALF // 2 - 1)
        def _(p):
            _finish(2 * p, rows0_v, sem0)
            _start(2 * p + 2, rows0_v, sem0)
            _finish(2 * p + 1, rows1_v, sem1)
            _start(2 * p + 3, rows1_v, sem1)

        _finish(HALF - 2, rows0_v, sem0)
        _finish(HALF - 1, rows1_v, sem1)

    plsc.subcore_barrier()

    base = sid * ROWS_PER_TILE
    pltpu.sync_copy(agg_sh.at[pl.ds(base, ROWS_PER_TILE)],
                    out_hbm.at[cid, pl.ds(base, ROWS_PER_TILE)])


ROW_BLK = 2000


def _scale_body(h0_ref, h1_ref, x_ref, y_ref, n_ref):
    deg = (h0_ref[...] + h1_ref[...]) * 0.5 + 1.0
    norm = lax.rsqrt(jnp.maximum(deg, 1e-12))
    n_ref[...] = norm
    y_ref[...] = x_ref[...] * norm


def _scale(h0, h1, x):
    return pl.pallas_call(
        _scale_body,
        out_shape=(jax.ShapeDtypeStruct((N_NODES, D), jnp.float32),
                   jax.ShapeDtypeStruct((N_NODES, 1), jnp.float32)),
        grid=(N_NODES // ROW_BLK,),
        in_specs=[
            pl.BlockSpec((ROW_BLK, 1), lambda i: (i, 0)),
            pl.BlockSpec((ROW_BLK, 1), lambda i: (i, 0)),
            pl.BlockSpec((ROW_BLK, D), lambda i: (i, 0)),
        ],
        out_specs=(pl.BlockSpec((ROW_BLK, D), lambda i: (i, 0)),
                   pl.BlockSpec((ROW_BLK, 1), lambda i: (i, 0))),
    )(h0, h1, x)


def _final_body(n_ref, y_ref, p0_ref, p1_ref, w_ref, b_ref, o_ref):
    agg = (p0_ref[...] + p1_ref[...] + y_ref[...]) * n_ref[...]
    out = jnp.dot(agg, w_ref[...], preferred_element_type=jnp.float32)
    o_ref[...] = jnp.maximum(out + b_ref[...], 0.0)


def _final(norm, y, p0, p1, W, b2):
    return pl.pallas_call(
        _final_body,
        out_shape=jax.ShapeDtypeStruct((N_NODES, D), jnp.float32),
        grid=(N_NODES // ROW_BLK,),
        in_specs=[
            pl.BlockSpec((ROW_BLK, 1), lambda i: (i, 0)),
            pl.BlockSpec((ROW_BLK, D), lambda i: (i, 0)),
            pl.BlockSpec((ROW_BLK, D), lambda i: (i, 0)),
            pl.BlockSpec((ROW_BLK, D), lambda i: (i, 0)),
            pl.BlockSpec((D, D), lambda i: (0, 0)),
            pl.BlockSpec((1, D), lambda i: (0, 0)),
        ],
        out_specs=pl.BlockSpec((ROW_BLK, D), lambda i: (i, 0)),
    )(norm, y, p0, p1, W, b2)


def kernel(x, edge_index, W, b):
    ei = edge_index.astype(jnp.int32)
    big = ei.reshape(2, NW, 2, E_CHUNKS // 2, E_CHUNK)

    hist = _hist_kernel(big)
    h0 = hist[:N_NODES].reshape(N_NODES, 1)
    h1 = hist[N_PAD:N_PAD + N_NODES].reshape(N_NODES, 1)

    y, norm = _scale(h0, h1, x)
    parts = _agg_kernel(y, big)

    return _final(norm, y, parts[0], parts[1], W, b.reshape(1, D))


# speedup vs baseline: 1.3095x; 1.0407x over previous
import functools

import jax
import jax.numpy as jnp
from jax import lax
from jax.experimental import pallas as pl
from jax.experimental.pallas import tpu as pltpu, tpu_sc as plsc

N_NODES = 10000
N_EDGES = 320000
D = 128

NC = 2
NS = 16
NW = NC * NS

H_CHUNK = 125
H_CHUNKS = (2 * N_EDGES) // NW // H_CHUNK
N_PAD = 640 * NS

_sc_mesh = plsc.VectorSubcoreMesh(core_axis_name="c", subcore_axis_name="s")


@functools.partial(
    pl.kernel,
    out_type=jax.ShapeDtypeStruct((NC * N_PAD,), jnp.float32),
    mesh=_sc_mesh,
    scratch_types=[
        pltpu.VMEM((H_CHUNKS // 4, H_CHUNK), jnp.int32),
        pltpu.VMEM((640,), jnp.float32),
        pltpu.VMEM_SHARED((N_PAD,), jnp.float32),
        pltpu.SemaphoreType.DMA,
    ],
)
def _hist_kernel(idx_hbm, out_hbm, idx_v, buf_v, hist_sh, hsem):
    cid = lax.axis_index("c")
    sid = lax.axis_index("s")
    wid = cid * NS + sid

    @pl.loop(0, 40)
    def _(i):
        buf_v[pl.ds(i * 16, 16)] = jnp.zeros((16,), jnp.float32)

    pltpu.sync_copy(buf_v, hist_sh.at[pl.ds(sid * 640, 640)])
    plsc.subcore_barrier()

    @pl.loop(0, 40)
    def _(i):
        buf_v[pl.ds(i * 16, 16)] = jnp.ones((16,), jnp.float32)

    GRP = 8
    for e in range(2):
        for h in range(2):
            pltpu.sync_copy(idx_hbm.at[e, wid, h], idx_v)

            @pl.loop(0, H_CHUNKS // 4 // GRP)
            def _(g):
                for j in range(GRP):
                    pltpu.async_copy(buf_v.at[pl.ds(0, H_CHUNK)],
                                     hist_sh.at[idx_v.at[g * GRP + j]],
                                     hsem, add=True)
                for j in range(GRP):
                    pltpu.make_async_copy(
                        buf_v.at[pl.ds(0, H_CHUNK)],
                        hist_sh.at[idx_v.at[g * GRP + j]], hsem).wait()

    plsc.subcore_barrier()
    pltpu.sync_copy(hist_sh.at[pl.ds(sid * 640, 640)], buf_v)
    pltpu.sync_copy(buf_v, out_hbm.at[pl.ds(cid * N_PAD + sid * 640, 640)])


E_CHUNK = 125
E_CHUNKS = N_EDGES // NW // E_CHUNK
ROWS_PER_TILE = N_PAD // NS
CP_CHUNK = 80


@functools.partial(
    pl.kernel,
    out_type=jax.ShapeDtypeStruct((NC, N_PAD, D), jnp.float32),
    mesh=_sc_mesh,
    scratch_types=[
        pltpu.VMEM((E_CHUNKS // 2, E_CHUNK), jnp.int32),
        pltpu.VMEM((E_CHUNKS // 2, E_CHUNK), jnp.int32),
        pltpu.VMEM((E_CHUNK, D), jnp.float32),
        pltpu.VMEM((E_CHUNK, D), jnp.float32),
        pltpu.VMEM_SHARED((N_PAD, D), jnp.float32),
        pltpu.SemaphoreType.DMA,
        pltpu.SemaphoreType.DMA,
    ],
)
def _agg_kernel(y_hbm, idx_hbm, out_hbm,
                src_v, dst_v, rows0_v, rows1_v, agg_sh, sem0, sem1):
    cid = lax.axis_index("c")
    sid = lax.axis_index("s")
    wid = cid * NS + sid

    @pl.loop(0, E_CHUNK)
    def _(r):
        @pl.loop(0, D // 16)
        def _(c):
            rows0_v[r, pl.ds(c * 16, 16)] = jnp.zeros((16,), jnp.float32)

    @pl.loop(0, ROWS_PER_TILE // CP_CHUNK)
    def _(k):
        pltpu.sync_copy(
            rows0_v.at[pl.ds(0, CP_CHUNK)],
            agg_sh.at[pl.ds(sid * ROWS_PER_TILE + k * CP_CHUNK, CP_CHUNK)])

    plsc.subcore_barrier()

    HALF = E_CHUNKS // 2

    def _start(j, buf, sem):
        pltpu.async_copy(y_hbm.at[src_v.at[j]], buf, sem)

    def _finish(j, buf, sem):
        pltpu.make_async_copy(y_hbm.at[src_v.at[j]], buf, sem).wait()
        pltpu.sync_copy(buf, agg_sh.at[dst_v.at[j]], add=True)

    for h in range(2):
        pltpu.sync_copy(idx_hbm.at[0, wid, h], src_v)
        pltpu.sync_copy(idx_hbm.at[1, wid, h], dst_v)

        _start(0, rows0_v, sem0)
        _start(1, rows1_v, sem1)

        @pl.loop(0, HALF // 2 - 1)
        def _(p):
            _finish(2 * p, rows0_v, sem0)
            _start(2 * p + 2, rows0_v, sem0)
            _finish(2 * p + 1, rows1_v, sem1)
            _start(2 * p + 3, rows1_v, sem1)

        _finish(HALF - 2, rows0_v, sem0)
        _finish(HALF - 1, rows1_v, sem1)

    plsc.subcore_barrier()

    base = sid * ROWS_PER_TILE
    pltpu.sync_copy(agg_sh.at[pl.ds(base, ROWS_PER_TILE)],
                    out_hbm.at[cid, pl.ds(base, ROWS_PER_TILE)])


ROW_BLK = 2000


def _scale_body(h0_ref, h1_ref, x_ref, y_ref, n_ref):
    deg = (h0_ref[...] + h1_ref[...]) * 0.5 + 1.0
    norm = lax.rsqrt(jnp.maximum(deg, 1e-12))
    n_ref[...] = norm
    y_ref[...] = x_ref[...] * norm


def _scale(h0, h1, x):
    return pl.pallas_call(
        _scale_body,
        out_shape=(jax.ShapeDtypeStruct((N_NODES, D), jnp.float32),
                   jax.ShapeDtypeStruct((N_NODES, 1), jnp.float32)),
        grid=(N_NODES // ROW_BLK,),
        in_specs=[
            pl.BlockSpec((ROW_BLK, 1), lambda i: (i, 0)),
            pl.BlockSpec((ROW_BLK, 1), lambda i: (i, 0)),
            pl.BlockSpec((ROW_BLK, D), lambda i: (i, 0)),
        ],
        out_specs=(pl.BlockSpec((ROW_BLK, D), lambda i: (i, 0)),
                   pl.BlockSpec((ROW_BLK, 1), lambda i: (i, 0))),
    )(h0, h1, x)


def _final_body(n_ref, y_ref, p_ref, w_ref, b_ref, o_ref):
    agg = (p_ref[0] + p_ref[1] + y_ref[...]) * n_ref[...]
    out = jnp.dot(agg, w_ref[...], preferred_element_type=jnp.float32)
    o_ref[...] = jnp.maximum(out + b_ref[...], 0.0)


def _final(norm, y, parts, W, b2):
    return pl.pallas_call(
        _final_body,
        out_shape=jax.ShapeDtypeStruct((N_NODES, D), jnp.float32),
        grid=(N_NODES // ROW_BLK,),
        in_specs=[
            pl.BlockSpec((ROW_BLK, 1), lambda i: (i, 0)),
            pl.BlockSpec((ROW_BLK, D), lambda i: (i, 0)),
            pl.BlockSpec((NC, ROW_BLK, D), lambda i: (0, i, 0)),
            pl.BlockSpec((D, D), lambda i: (0, 0)),
            pl.BlockSpec((1, D), lambda i: (0, 0)),
        ],
        out_specs=pl.BlockSpec((ROW_BLK, D), lambda i: (i, 0)),
    )(norm, y, parts, W, b2)


def kernel(x, edge_index, W, b):
    ei = edge_index.astype(jnp.int32)
    big = ei.reshape(2, NW, 2, E_CHUNKS // 2, E_CHUNK)

    hist = _hist_kernel(big)
    h0 = hist[:N_NODES].reshape(N_NODES, 1)
    h1 = hist[N_PAD:N_PAD + N_NODES].reshape(N_NODES, 1)

    y, norm = _scale(h0, h1, x)
    parts = _agg_kernel(y, big)

    return _final(norm, y, parts, W, b.reshape(1, D))
